# baseline (speedup 1.0000x reference)
import functools

import jax
import jax.numpy as jnp
from jax import lax
from jax.experimental import pallas as pl
from jax.experimental.pallas import tpu as pltpu
from jax.experimental.pallas import tpu_sc as plsc

B = 128
VOCAB = 100000
TOPK = 5
LANES = 16

RB = 8
NRB = B // RB


def _gather_body(yp_hbm, y_hbm, g_hbm, ys, buf, sem):
    c = lax.axis_index("c")

    @pl.when(c == 0)
    def _():
        pltpu.sync_copy(y_hbm, ys)
        copies = []
        for b in range(B):
            start = (ys[b] // LANES) * LANES
            copies.append(
                pltpu.async_copy(
                    yp_hbm.at[b, pl.ds(start, LANES)], buf.at[b], sem
                )
            )
        for cp in copies:
            cp.wait()
        pltpu.sync_copy(buf, g_hbm)


@functools.cache
def _gather_segments():
    return pl.kernel(
        _gather_body,
        out_type=jax.ShapeDtypeStruct((B, LANES), jnp.float32),
        mesh=plsc.ScalarSubcoreMesh(axis_name="c", num_cores=2),
        scratch_types=[
            pltpu.SMEM((B,), jnp.int32),
            pltpu.VMEM_SHARED((B, LANES), jnp.float32),
            pltpu.SemaphoreType.DMA,
        ],
    )


def _scan_body(g_ref, y_ref, x_ref, out_ref, acc_ref):
    i = pl.program_id(0)
    yy = y_ref[...]
    seg_lane = lax.broadcasted_iota(jnp.int32, (RB, LANES), 1)
    t = jnp.sum(
        jnp.where(seg_lane == yy % LANES, g_ref[...], 0.0),
        axis=1,
        keepdims=True,
    )
    vals = x_ref[...]
    col = lax.broadcasted_iota(jnp.int32, (RB, VOCAB), 1)
    m = (vals > t) | ((vals == t) & (col < yy))
    acc_ref[pl.ds(i * RB, RB), :] = jnp.sum(
        m.astype(jnp.int32), axis=1, keepdims=True
    )

    @pl.when(i == NRB - 1)
    def _():
        out_ref[...] = jnp.sum(
            (acc_ref[...] < TOPK).astype(jnp.int32), axis=(0, 1), keepdims=True
        )


def _count_hits(y_pred, g, y):
    return pl.pallas_call(
        _scan_body,
        grid=(NRB,),
        in_specs=[
            pl.BlockSpec((RB, LANES), lambda i: (i, 0)),
            pl.BlockSpec((RB, 1), lambda i: (i, 0)),
            pl.BlockSpec((RB, VOCAB), lambda i: (i, 0)),
        ],
        out_specs=pl.BlockSpec((1, 1), lambda i: (0, 0)),
        out_shape=jax.ShapeDtypeStruct((1, 1), jnp.int32),
        scratch_shapes=[
            pltpu.VMEM((B, 1), jnp.int32),
        ],
    )(g, y.reshape(B, 1), y_pred)


def kernel(y_pred, y):
    y32 = y.astype(jnp.int32)
    g = _gather_segments()(y_pred, y32)
    return _count_hits(y_pred, g, y32)[0, 0]
